# baseline (speedup 1.0000x reference)
from functools import partial

import numpy as np
import jax
import jax.numpy as jnp
from jax.experimental import pallas as pl
from jax.experimental.pallas import tpu as pltpu

_EPS = 1e-4
_VMEM_LIMIT = 96 * 1024 * 1024


def _norm_weight(weight, gain):
    w = weight.astype(jnp.float32)
    fan_in = int(np.prod(w.shape[1:]))
    norm = jnp.sqrt(jnp.sum(w * w, axis=tuple(range(1, w.ndim)), keepdims=True))
    norm = _EPS + norm * (1.0 / np.sqrt(fan_in))
    return (w / norm) * (float(gain) / np.sqrt(fan_in))


def _conv_kernel(x_ref, w_ref, o_ref, *, h, w, k, pad):
    cin = x_ref.shape[1]
    hw = h * w
    xb = x_ref[0].astype(jnp.bfloat16)
    side = pad * w + pad
    xp = jnp.pad(xb, ((0, 0), (side, side)))
    col = jax.lax.broadcasted_iota(jnp.int32, (1, hw), 1) % w
    pieces = []
    for dy in range(k):
        for dx in range(k):
            off = side + (dy - pad) * w + (dx - pad)
            s = xp[:, off:off + hw]
            d = dx - pad
            if d < 0:
                s = jnp.where(col >= -d, s, jnp.bfloat16(0))
            elif d > 0:
                s = jnp.where(col < w - d, s, jnp.bfloat16(0))
            pieces.append(s)
    patches = jnp.concatenate(pieces, axis=0)
    o_ref[0] = jnp.dot(w_ref[...], patches, preferred_element_type=jnp.float32)


def kernel(x, weight):
    n, cin, h, w = x.shape
    cout, cin_w, kh, kw = weight.shape
    assert cin == cin_w and kh == kw and kh % 2 == 1
    k = kh
    pad = k // 2
    hw = h * w

    wn = _norm_weight(weight, 1.0)
    w2 = jnp.transpose(wn, (0, 2, 3, 1)).reshape(cout, k * k * cin)
    w2 = w2.astype(jnp.bfloat16)
    xf = x.reshape(n, cin, hw)

    cost = pl.CostEstimate(
        flops=2 * n * hw * k * k * cin * cout,
        transcendentals=0,
        bytes_accessed=(xf.size * 4 + w2.size * 2 + n * cout * hw * 4))

    out = pl.pallas_call(
        partial(_conv_kernel, h=h, w=w, k=k, pad=pad),
        out_shape=jax.ShapeDtypeStruct((n, cout, hw), jnp.float32),
        grid=(n,),
        in_specs=[
            pl.BlockSpec((1, cin, hw), lambda b: (b, 0, 0)),
            pl.BlockSpec((cout, k * k * cin), lambda b: (0, 0)),
        ],
        out_specs=pl.BlockSpec((1, cout, hw), lambda b: (b, 0, 0)),
        compiler_params=pltpu.CompilerParams(
            dimension_semantics=("parallel",),
            vmem_limit_bytes=_VMEM_LIMIT),
        cost_estimate=cost,
    )(xf, w2)
    return out.reshape(n, cout, h, w)
